# baseline (device time: 225126 ns/iter reference)
import jax
import jax.numpy as jnp
from jax import lax
from jax.experimental import pallas as pl
from jax.experimental.pallas import tpu as pltpu

N_DEV = 8
SQ = 1024
SKV = 1024
HQ = 8
DH = 128
DMODEL = 1024
BLK = 64
SCALE = 0.08838834764831843


def kernel(x, Wq, K_ext, V_ext, Wo):
    my = lax.axis_index("i")

    xb = x[0].astype(jnp.bfloat16)
    wq = Wq.astype(jnp.bfloat16)
    k = lax.dynamic_slice_in_dim(K_ext[0], my * HQ, HQ, axis=1)
    k = jnp.transpose(k, (1, 0, 2)).astype(jnp.bfloat16)
    v = lax.dynamic_slice_in_dim(V_ext[0], my * HQ, HQ, axis=1)
    v = jnp.transpose(v, (1, 0, 2)).astype(jnp.bfloat16)
    wo = Wo.astype(jnp.bfloat16)

    def body(x_ref, wq_ref, k_ref, v_ref, wo_ref, out_ref,
             part_ref, comm_ref, send_sems, recv_sems):
        me = lax.axis_index("i")
        right = lax.rem(me + 1, N_DEV)
        left = lax.rem(me + N_DEV - 1, N_DEV)

        barrier_sem = pltpu.get_barrier_semaphore()
        for nbr in (left, right):
            pl.semaphore_signal(
                barrier_sem, inc=1,
                device_id=(nbr,), device_id_type=pl.DeviceIdType.MESH,
            )
        pl.semaphore_wait(barrier_sem, 2)

        xv = x_ref[...]
        qb = lax.broadcasted_iota(jnp.int32, (SQ, SKV), 0) // BLK
        kb = lax.broadcasted_iota(jnp.int32, (SQ, SKV), 1) // BLK
        mask = (qb == kb) | (kb == 0) | ((qb + kb) % 3 == 0)

        acc = jnp.zeros((SQ, DMODEL), jnp.float32)
        for h in range(HQ):
            q = jnp.dot(
                xv, wq_ref[:, h * DH:(h + 1) * DH],
                preferred_element_type=jnp.float32,
            ).astype(jnp.bfloat16)
            s = lax.dot_general(
                q, k_ref[h], (((1,), (1,)), ((), ())),
                preferred_element_type=jnp.float32,
            ) * SCALE
            s = jnp.where(mask, s, -1e9)
            s = s - jnp.max(s, axis=1, keepdims=True)
            w = jnp.exp(s)
            w = (w / jnp.sum(w, axis=1, keepdims=True)).astype(jnp.bfloat16)
            ctx = jnp.dot(
                w, v_ref[h], preferred_element_type=jnp.float32
            ).astype(jnp.bfloat16)
            acc = acc + jnp.dot(
                ctx, wo_ref[h * DH:(h + 1) * DH, :],
                preferred_element_type=jnp.float32,
            )

        out_ref[...] = acc
        part_ref[...] = acc.astype(jnp.bfloat16)

        for h in range(N_DEV - 1):
            src = part_ref if h == 0 else comm_ref.at[h - 1]
            rdma = pltpu.make_async_remote_copy(
                src_ref=src,
                dst_ref=comm_ref.at[h],
                send_sem=send_sems.at[h],
                recv_sem=recv_sems.at[h],
                device_id=(right,),
                device_id_type=pl.DeviceIdType.MESH,
            )
            rdma.start()
            rdma.wait()
            out_ref[...] += comm_ref[h].astype(jnp.float32)

    out = pl.pallas_call(
        body,
        out_shape=jax.ShapeDtypeStruct((SQ, DMODEL), jnp.float32),
        in_specs=[pl.BlockSpec(memory_space=pltpu.VMEM)] * 5,
        out_specs=pl.BlockSpec(memory_space=pltpu.VMEM),
        scratch_shapes=[
            pltpu.VMEM((SQ, DMODEL), jnp.bfloat16),
            pltpu.VMEM((N_DEV - 1, SQ, DMODEL), jnp.bfloat16),
            pltpu.SemaphoreType.DMA((N_DEV - 1,)),
            pltpu.SemaphoreType.DMA((N_DEV - 1,)),
        ],
        compiler_params=pltpu.CompilerParams(collective_id=0),
    )(xb, wq, k, v, wo)

    return out[None]


# device time: 105854 ns/iter; 2.1268x vs baseline; 2.1268x over previous
import functools

import jax
import jax.numpy as jnp
from jax import lax
from jax.experimental import pallas as pl
from jax.experimental.pallas import tpu as pltpu

N_DEV = 8
SQ = 1024
SKV = 1024
HQ = 8
DH = 128
DMODEL = 1024
BLK = 64
SCALE = 0.08838834764831843


def kernel(x, Wq, K_ext, V_ext, Wo):
    my = lax.axis_index("i")

    xb = x[0].astype(jnp.bfloat16)
    wq = Wq.astype(jnp.bfloat16)
    k = lax.dynamic_slice_in_dim(K_ext[0], my * HQ, HQ, axis=1)
    k = jnp.transpose(k, (1, 0, 2)).astype(jnp.bfloat16)
    v = lax.dynamic_slice_in_dim(V_ext[0], my * HQ, HQ, axis=1)
    v = jnp.transpose(v, (1, 0, 2)).astype(jnp.bfloat16)
    wo = Wo.astype(jnp.bfloat16)

    def body(x_ref, wq_ref, k_ref, v_ref, wo_ref, out_ref,
             acc_ref, res_ref,
             sb0, sb1, sb2, rb0, rb1, rb2,
             rs_send_sems, rs_recv_sems, ag_send_sems, ag_recv_sems):
        me = lax.axis_index("i")
        partners = [me ^ 1, me ^ 2, me ^ 4]

        barrier_sem = pltpu.get_barrier_semaphore()
        for p in partners:
            pl.semaphore_signal(
                barrier_sem, inc=1,
                device_id=(p,), device_id_type=pl.DeviceIdType.MESH,
            )
        pl.semaphore_wait(barrier_sem, 3)

        xv = x_ref[...]
        qb = lax.broadcasted_iota(jnp.int32, (SQ, SKV), 0) // BLK
        kb = lax.broadcasted_iota(jnp.int32, (SQ, SKV), 1) // BLK
        mask = (qb == kb) | (kb == 0) | ((qb + kb) % 3 == 0)

        acc = jnp.zeros((SQ, DMODEL), jnp.float32)
        for h in range(HQ):
            q = jnp.dot(
                xv, wq_ref[:, h * DH:(h + 1) * DH],
                preferred_element_type=jnp.float32,
            ).astype(jnp.bfloat16)
            s = lax.dot_general(
                q, k_ref[h], (((1,), (1,)), ((), ())),
                preferred_element_type=jnp.float32,
            ) * SCALE
            s = jnp.where(mask, s, -1e9)
            s = s - jnp.max(s, axis=1, keepdims=True)
            w = jnp.exp(s)
            w = (w / jnp.sum(w, axis=1, keepdims=True)).astype(jnp.bfloat16)
            ctx = jnp.dot(
                w, v_ref[h], preferred_element_type=jnp.float32
            ).astype(jnp.bfloat16)
            acc = acc + jnp.dot(
                ctx, wo_ref[h * DH:(h + 1) * DH, :],
                preferred_element_type=jnp.float32,
            )
        acc_ref[...] = acc

        send_bufs = [sb0, sb1, sb2]
        recv_bufs = [rb0, rb1, rb2]
        region_start = jnp.int32(0)
        for s in range(3):
            half = SQ >> (s + 1)
            bit = (me >> s) & 1
            keep_start = region_start + bit * half
            send_start = region_start + (1 - bit) * half
            partner = me ^ (1 << s)

            send_bufs[s][...] = acc_ref[
                pl.ds(send_start, half), :
            ].astype(jnp.bfloat16)
            rdma = pltpu.make_async_remote_copy(
                src_ref=send_bufs[s],
                dst_ref=recv_bufs[s],
                send_sem=rs_send_sems.at[s],
                recv_sem=rs_recv_sems.at[s],
                device_id=(partner,),
                device_id_type=pl.DeviceIdType.MESH,
            )
            rdma.start()
            rdma.wait()
            acc_ref[pl.ds(keep_start, half), :] += recv_bufs[s][
                ...
            ].astype(jnp.float32)
            region_start = keep_start

        res_ref[pl.ds(region_start, SQ // N_DEV), :] = acc_ref[
            pl.ds(region_start, SQ // N_DEV), :
        ].astype(jnp.bfloat16)

        own_start = region_start
        own_size = SQ // N_DEV
        for s in (2, 1, 0):
            partner = me ^ (1 << s)
            rdma = pltpu.make_async_remote_copy(
                src_ref=res_ref.at[pl.ds(own_start, own_size), :],
                dst_ref=res_ref.at[pl.ds(own_start, own_size), :],
                send_sem=ag_send_sems.at[s],
                recv_sem=ag_recv_sems.at[s],
                device_id=(partner,),
                device_id_type=pl.DeviceIdType.MESH,
            )
            rdma.start()
            rdma.wait()
            bit = (me >> s) & 1
            own_start = own_start - bit * own_size
            own_size = own_size * 2

        out_ref[...] = res_ref[...].astype(jnp.float32)

        @functools.partial(
            pl.run_scoped, second_barrier=pltpu.SemaphoreType.REGULAR
        )
        def _(second_barrier):
            for p in partners:
                pl.semaphore_signal(
                    second_barrier, inc=1,
                    device_id=(p,), device_id_type=pl.DeviceIdType.MESH,
                )
            pl.semaphore_wait(second_barrier, 3)

    out = pl.pallas_call(
        body,
        out_shape=jax.ShapeDtypeStruct((SQ, DMODEL), jnp.float32),
        in_specs=[pl.BlockSpec(memory_space=pltpu.VMEM)] * 5,
        out_specs=pl.BlockSpec(memory_space=pltpu.VMEM),
        scratch_shapes=[
            pltpu.VMEM((SQ, DMODEL), jnp.float32),
            pltpu.VMEM((SQ, DMODEL), jnp.bfloat16),
            pltpu.VMEM((SQ // 2, DMODEL), jnp.bfloat16),
            pltpu.VMEM((SQ // 4, DMODEL), jnp.bfloat16),
            pltpu.VMEM((SQ // 8, DMODEL), jnp.bfloat16),
            pltpu.VMEM((SQ // 2, DMODEL), jnp.bfloat16),
            pltpu.VMEM((SQ // 4, DMODEL), jnp.bfloat16),
            pltpu.VMEM((SQ // 8, DMODEL), jnp.bfloat16),
            pltpu.SemaphoreType.DMA((3,)),
            pltpu.SemaphoreType.DMA((3,)),
            pltpu.SemaphoreType.DMA((3,)),
            pltpu.SemaphoreType.DMA((3,)),
        ],
        compiler_params=pltpu.CompilerParams(collective_id=0),
    )(xb, wq, k, v, wo)

    return out[None]


# device time: 91133 ns/iter; 2.4703x vs baseline; 1.1615x over previous
import functools

import jax
import jax.numpy as jnp
from jax import lax
from jax.experimental import pallas as pl
from jax.experimental.pallas import tpu as pltpu

N_DEV = 8
SQ = 1024
SKV = 1024
HQ = 8
DH = 128
DMODEL = 1024
BLK = 64
SCALE = 0.08838834764831843
HALF = SQ // 2


def kernel(x, Wq, K_ext, V_ext, Wo):
    my = lax.axis_index("i")

    xb = x[0].astype(jnp.bfloat16)
    wq = Wq.astype(jnp.bfloat16)
    k = lax.dynamic_slice_in_dim(K_ext[0], my * HQ, HQ, axis=1)
    k = jnp.transpose(k, (1, 0, 2)).astype(jnp.bfloat16)
    v = lax.dynamic_slice_in_dim(V_ext[0], my * HQ, HQ, axis=1)
    v = jnp.transpose(v, (1, 0, 2)).astype(jnp.bfloat16)
    wo = Wo.astype(jnp.bfloat16)

    def body(x_ref, wq_ref, k_ref, v_ref, wo_ref, out_ref,
             acc_ref, sb0, sb1, sb2, rb0, rb1, rb2,
             rs_send_sems, rs_recv_sems, ag_send_sems, ag_recv_sems):
        me = lax.axis_index("i")
        partners = [me ^ 1, me ^ 2, me ^ 4]

        barrier_sem = pltpu.get_barrier_semaphore()
        for p in partners:
            pl.semaphore_signal(
                barrier_sem, inc=1,
                device_id=(p,), device_id_type=pl.DeviceIdType.MESH,
            )
        pl.semaphore_wait(barrier_sem, 3)

        def compute_rows(row0):
            xv = x_ref[pl.ds(row0, HALF), :]
            qb = (lax.broadcasted_iota(jnp.int32, (HALF, SKV), 0) + row0) // BLK
            kb = lax.broadcasted_iota(jnp.int32, (HALF, SKV), 1) // BLK
            mask = (qb == kb) | (kb == 0) | ((qb + kb) % 3 == 0)
            acc = jnp.zeros((HALF, DMODEL), jnp.float32)
            for h in range(HQ):
                q = jnp.dot(
                    xv, wq_ref[:, h * DH:(h + 1) * DH],
                    preferred_element_type=jnp.float32,
                ).astype(jnp.bfloat16)
                s = lax.dot_general(
                    q, k_ref[h], (((1,), (1,)), ((), ())),
                    preferred_element_type=jnp.float32,
                ) * SCALE
                w = jnp.exp(jnp.where(mask, s, -1e9))
                w = (w * (1.0 / jnp.sum(w, axis=1, keepdims=True))).astype(
                    jnp.bfloat16
                )
                ctx = jnp.dot(
                    w, v_ref[h], preferred_element_type=jnp.float32
                ).astype(jnp.bfloat16)
                acc = acc + jnp.dot(
                    ctx, wo_ref[h * DH:(h + 1) * DH, :],
                    preferred_element_type=jnp.float32,
                )
            return acc

        bit0 = me & 1
        abs_keep = bit0 * HALF
        sb0[...] = compute_rows((1 - bit0) * HALF).astype(jnp.bfloat16)
        rdma0 = pltpu.make_async_remote_copy(
            src_ref=sb0, dst_ref=rb0,
            send_sem=rs_send_sems.at[0], recv_sem=rs_recv_sems.at[0],
            device_id=(me ^ 1,), device_id_type=pl.DeviceIdType.MESH,
        )
        rdma0.start()
        keep = compute_rows(abs_keep)
        rdma0.wait()
        acc_ref[...] = keep + rb0[...].astype(jnp.float32)

        send_bufs = {1: sb1, 2: sb2}
        recv_bufs = {1: rb1, 2: rb2}
        l = jnp.int32(0)
        for s in (1, 2):
            half = HALF >> s
            bit = (me >> s) & 1
            l_keep = l + bit * half
            l_send = l + (1 - bit) * half
            send_bufs[s][...] = acc_ref[pl.ds(l_send, half), :].astype(
                jnp.bfloat16
            )
            rdma = pltpu.make_async_remote_copy(
                src_ref=send_bufs[s], dst_ref=recv_bufs[s],
                send_sem=rs_send_sems.at[s], recv_sem=rs_recv_sems.at[s],
                device_id=(me ^ (1 << s),),
                device_id_type=pl.DeviceIdType.MESH,
            )
            rdma.start()
            rdma.wait()
            acc_ref[pl.ds(l_keep, half), :] += recv_bufs[s][...].astype(
                jnp.float32
            )
            l = l_keep

        own_size = SQ // N_DEV
        own_start = abs_keep + l
        out_ref[pl.ds(own_start, own_size), :] = acc_ref[
            pl.ds(l, own_size), :
        ].astype(jnp.bfloat16)

        for s in (2, 1, 0):
            rdma = pltpu.make_async_remote_copy(
                src_ref=out_ref.at[pl.ds(own_start, own_size), :],
                dst_ref=out_ref.at[pl.ds(own_start, own_size), :],
                send_sem=ag_send_sems.at[s], recv_sem=ag_recv_sems.at[s],
                device_id=(me ^ (1 << s),),
                device_id_type=pl.DeviceIdType.MESH,
            )
            rdma.start()
            rdma.wait()
            bit = (me >> s) & 1
            own_start = own_start - bit * own_size
            own_size = own_size * 2

        @functools.partial(
            pl.run_scoped, second_barrier=pltpu.SemaphoreType.REGULAR
        )
        def _(second_barrier):
            for p in partners:
                pl.semaphore_signal(
                    second_barrier, inc=1,
                    device_id=(p,), device_id_type=pl.DeviceIdType.MESH,
                )
            pl.semaphore_wait(second_barrier, 3)

    out = pl.pallas_call(
        body,
        out_shape=jax.ShapeDtypeStruct((SQ, DMODEL), jnp.bfloat16),
        in_specs=[pl.BlockSpec(memory_space=pltpu.VMEM)] * 5,
        out_specs=pl.BlockSpec(memory_space=pltpu.VMEM),
        scratch_shapes=[
            pltpu.VMEM((HALF, DMODEL), jnp.float32),
            pltpu.VMEM((HALF, DMODEL), jnp.bfloat16),
            pltpu.VMEM((HALF // 2, DMODEL), jnp.bfloat16),
            pltpu.VMEM((HALF // 4, DMODEL), jnp.bfloat16),
            pltpu.VMEM((HALF, DMODEL), jnp.bfloat16),
            pltpu.VMEM((HALF // 2, DMODEL), jnp.bfloat16),
            pltpu.VMEM((HALF // 4, DMODEL), jnp.bfloat16),
            pltpu.SemaphoreType.DMA((3,)),
            pltpu.SemaphoreType.DMA((3,)),
            pltpu.SemaphoreType.DMA((3,)),
            pltpu.SemaphoreType.DMA((3,)),
        ],
        compiler_params=pltpu.CompilerParams(collective_id=0),
    )(xb, wq, k, v, wo)

    return out[None]


# device time: 90269 ns/iter; 2.4939x vs baseline; 1.0096x over previous
import functools

import jax
import jax.numpy as jnp
from jax import lax
from jax.experimental import pallas as pl
from jax.experimental.pallas import tpu as pltpu

N_DEV = 8
SQ = 1024
SKV = 1024
HQ = 8
DH = 128
DMODEL = 1024
BLK = 64
SCALE = 0.08838834764831843
HALF = SQ // 2


def kernel(x, Wq, K_ext, V_ext, Wo):
    my = lax.axis_index("i")

    xb = x[0].astype(jnp.bfloat16)
    wq = Wq.astype(jnp.bfloat16)
    k = lax.dynamic_slice_in_dim(K_ext[0], my * HQ, HQ, axis=1)
    k = jnp.transpose(k, (1, 0, 2)).astype(jnp.bfloat16)
    v = lax.dynamic_slice_in_dim(V_ext[0], my * HQ, HQ, axis=1)
    v = jnp.transpose(v, (1, 0, 2)).astype(jnp.bfloat16)
    wo = Wo.astype(jnp.bfloat16)

    def body(x_ref, wq_ref, k_ref, v_ref, wo_ref, out_ref,
             acc_ref, sb0, sb1, sb2, rb0, rb1, rb2,
             rs_send_sems, rs_recv_sems, ag_send_sems, ag_recv_sems):
        me = lax.axis_index("i")
        partners = [me ^ 1, me ^ 2, me ^ 4]

        barrier_sem = pltpu.get_barrier_semaphore()
        for p in partners:
            pl.semaphore_signal(
                barrier_sem, inc=1,
                device_id=(p,), device_id_type=pl.DeviceIdType.MESH,
            )
        pl.semaphore_wait(barrier_sem, 3)

        def compute_rows(row0):
            xv = x_ref[pl.ds(row0, HALF), :]
            qb = (lax.broadcasted_iota(jnp.int32, (HALF, SKV), 0) + row0) // BLK
            kb = lax.broadcasted_iota(jnp.int32, (HALF, SKV), 1) // BLK
            mask = (qb == kb) | (kb == 0) | ((qb + kb) % 3 == 0)
            acc = jnp.zeros((HALF, DMODEL), jnp.float32)
            for h in range(HQ):
                q = (jnp.dot(
                    xv, wq_ref[:, h * DH:(h + 1) * DH],
                    preferred_element_type=jnp.float32,
                ) * SCALE).astype(jnp.bfloat16)
                s = lax.dot_general(
                    q, k_ref[h], (((1,), (1,)), ((), ())),
                    preferred_element_type=jnp.float32,
                ).astype(jnp.bfloat16)
                w = jnp.exp(jnp.where(mask, s, jnp.bfloat16(-1e9)))
                wsum = jnp.sum(w, axis=1, keepdims=True, dtype=jnp.float32)
                w = w * (1.0 / wsum).astype(jnp.bfloat16)
                ctx = jnp.dot(
                    w, v_ref[h], preferred_element_type=jnp.float32
                ).astype(jnp.bfloat16)
                acc = acc + jnp.dot(
                    ctx, wo_ref[h * DH:(h + 1) * DH, :],
                    preferred_element_type=jnp.float32,
                )
            return acc

        bit0 = me & 1
        abs_keep = bit0 * HALF
        sb0[...] = compute_rows((1 - bit0) * HALF).astype(jnp.bfloat16)
        rdma0 = pltpu.make_async_remote_copy(
            src_ref=sb0, dst_ref=rb0,
            send_sem=rs_send_sems.at[0], recv_sem=rs_recv_sems.at[0],
            device_id=(me ^ 1,), device_id_type=pl.DeviceIdType.MESH,
        )
        rdma0.start()
        keep = compute_rows(abs_keep)
        rdma0.wait()
        acc_ref[...] = keep + rb0[...].astype(jnp.float32)

        send_bufs = {1: sb1, 2: sb2}
        recv_bufs = {1: rb1, 2: rb2}
        l = jnp.int32(0)
        for s in (1, 2):
            half = HALF >> s
            bit = (me >> s) & 1
            l_keep = l + bit * half
            l_send = l + (1 - bit) * half
            send_bufs[s][...] = acc_ref[pl.ds(l_send, half), :].astype(
                jnp.bfloat16
            )
            rdma = pltpu.make_async_remote_copy(
                src_ref=send_bufs[s], dst_ref=recv_bufs[s],
                send_sem=rs_send_sems.at[s], recv_sem=rs_recv_sems.at[s],
                device_id=(me ^ (1 << s),),
                device_id_type=pl.DeviceIdType.MESH,
            )
            rdma.start()
            rdma.wait()
            acc_ref[pl.ds(l_keep, half), :] += recv_bufs[s][...].astype(
                jnp.float32
            )
            l = l_keep

        own_size = SQ // N_DEV
        own_start = abs_keep + l
        out_ref[pl.ds(own_start, own_size), :] = acc_ref[
            pl.ds(l, own_size), :
        ].astype(jnp.bfloat16)

        for s in (2, 1, 0):
            rdma = pltpu.make_async_remote_copy(
                src_ref=out_ref.at[pl.ds(own_start, own_size), :],
                dst_ref=out_ref.at[pl.ds(own_start, own_size), :],
                send_sem=ag_send_sems.at[s], recv_sem=ag_recv_sems.at[s],
                device_id=(me ^ (1 << s),),
                device_id_type=pl.DeviceIdType.MESH,
            )
            rdma.start()
            rdma.wait()
            bit = (me >> s) & 1
            own_start = own_start - bit * own_size
            own_size = own_size * 2

        @functools.partial(
            pl.run_scoped, second_barrier=pltpu.SemaphoreType.REGULAR
        )
        def _(second_barrier):
            for p in partners:
                pl.semaphore_signal(
                    second_barrier, inc=1,
                    device_id=(p,), device_id_type=pl.DeviceIdType.MESH,
                )
            pl.semaphore_wait(second_barrier, 3)

    out = pl.pallas_call(
        body,
        out_shape=jax.ShapeDtypeStruct((SQ, DMODEL), jnp.bfloat16),
        in_specs=[pl.BlockSpec(memory_space=pltpu.VMEM)] * 5,
        out_specs=pl.BlockSpec(memory_space=pltpu.VMEM),
        scratch_shapes=[
            pltpu.VMEM((HALF, DMODEL), jnp.float32),
            pltpu.VMEM((HALF, DMODEL), jnp.bfloat16),
            pltpu.VMEM((HALF // 2, DMODEL), jnp.bfloat16),
            pltpu.VMEM((HALF // 4, DMODEL), jnp.bfloat16),
            pltpu.VMEM((HALF, DMODEL), jnp.bfloat16),
            pltpu.VMEM((HALF // 2, DMODEL), jnp.bfloat16),
            pltpu.VMEM((HALF // 4, DMODEL), jnp.bfloat16),
            pltpu.SemaphoreType.DMA((3,)),
            pltpu.SemaphoreType.DMA((3,)),
            pltpu.SemaphoreType.DMA((3,)),
            pltpu.SemaphoreType.DMA((3,)),
        ],
        compiler_params=pltpu.CompilerParams(collective_id=0),
    )(xb, wq, k, v, wo)

    return out[None]


# device time: 83100 ns/iter; 2.7091x vs baseline; 1.0863x over previous
import functools

import jax
import jax.numpy as jnp
from jax import lax
from jax.experimental import pallas as pl
from jax.experimental.pallas import tpu as pltpu

N_DEV = 8
SQ = 1024
SKV = 1024
HQ = 8
DH = 128
DMODEL = 1024
BLK = 64
SCALE = 0.08838834764831843
QTR = SQ // 4
CHUNK = SQ // N_DEV


def kernel(x, Wq, K_ext, V_ext, Wo):
    my = lax.axis_index("i")

    xb = x[0].astype(jnp.bfloat16)
    wq = Wq.astype(jnp.bfloat16)
    k = lax.dynamic_slice_in_dim(K_ext[0], my * HQ, HQ, axis=1)
    k = jnp.transpose(k, (1, 0, 2)).astype(jnp.bfloat16)
    v = lax.dynamic_slice_in_dim(V_ext[0], my * HQ, HQ, axis=1)
    v = jnp.transpose(v, (1, 0, 2)).astype(jnp.bfloat16)
    wo = Wo.astype(jnp.bfloat16)

    def body(x_ref, wq_ref, k_ref, v_ref, wo_ref, out_ref,
             keep_ref, sb0a, sb0b, sb1, sb2, rb0a, rb0b, rb1, rb2,
             rs_send_sems, rs_recv_sems, ag_send_sems, ag_recv_sems):
        me = lax.axis_index("i")
        partners = [me ^ 1, me ^ 2, me ^ 4]

        barrier_sem = pltpu.get_barrier_semaphore()
        for p in partners:
            pl.semaphore_signal(
                barrier_sem, inc=1,
                device_id=(p,), device_id_type=pl.DeviceIdType.MESH,
            )
        pl.semaphore_wait(barrier_sem, 3)

        def compute_rows(row0):
            xv = x_ref[pl.ds(row0, QTR), :]
            qb = (lax.broadcasted_iota(jnp.int32, (QTR, SKV), 0) + row0) // BLK
            kb = lax.broadcasted_iota(jnp.int32, (QTR, SKV), 1) // BLK
            mask = (qb == kb) | (kb == 0) | ((qb + kb) % 3 == 0)
            acc = jnp.zeros((QTR, DMODEL), jnp.float32)
            for h in range(HQ):
                q = (jnp.dot(
                    xv, wq_ref[:, h * DH:(h + 1) * DH],
                    preferred_element_type=jnp.float32,
                ) * SCALE).astype(jnp.bfloat16)
                s = lax.dot_general(
                    q, k_ref[h], (((1,), (1,)), ((), ())),
                    preferred_element_type=jnp.float32,
                ).astype(jnp.bfloat16)
                w = jnp.exp(jnp.where(mask, s, jnp.bfloat16(-1e9)))
                wsum = jnp.sum(w, axis=1, keepdims=True, dtype=jnp.float32)
                w = w * (1.0 / wsum).astype(jnp.bfloat16)
                ctx = jnp.dot(
                    w, v_ref[h], preferred_element_type=jnp.float32
                ).astype(jnp.bfloat16)
                acc = acc + jnp.dot(
                    ctx, wo_ref[h * DH:(h + 1) * DH, :],
                    preferred_element_type=jnp.float32,
                )
            return acc

        def exchange(src, dst, sem_slot, partner, sems=None):
            r = pltpu.make_async_remote_copy(
                src_ref=src, dst_ref=dst,
                send_sem=(sems or (rs_send_sems, rs_recv_sems))[0].at[sem_slot],
                recv_sem=(sems or (rs_send_sems, rs_recv_sems))[1].at[sem_slot],
                device_id=(partner,), device_id_type=pl.DeviceIdType.MESH,
            )
            r.start()
            return r

        bit0 = me & 1
        bit1 = (me >> 1) & 1
        bit2 = (me >> 2) & 1
        abs_keep = bit0 * (SQ // 2)
        abs_send = (1 - bit0) * (SQ // 2)
        q_fwd = (1 - bit1) * QTR
        q_keep = bit1 * QTR

        sb0a[...] = compute_rows(abs_send + q_fwd).astype(jnp.bfloat16)
        r0a = exchange(sb0a, rb0a, 0, me ^ 1)
        sb0b[...] = compute_rows(abs_send + q_keep).astype(jnp.bfloat16)
        r0b = exchange(sb0b, rb0b, 1, me ^ 1)

        c = compute_rows(abs_keep + q_fwd)
        r0a.wait()
        sb1[...] = (c + rb0a[...].astype(jnp.float32)).astype(jnp.bfloat16)
        r1 = exchange(sb1, rb1, 2, me ^ 2)

        d = compute_rows(abs_keep + q_keep)
        r0b.wait()
        keep = d + rb0b[...].astype(jnp.float32)
        r1.wait()
        keep_ref[...] = keep + rb1[...].astype(jnp.float32)

        o_send = (1 - bit2) * CHUNK
        o_keep = bit2 * CHUNK
        sb2[...] = keep_ref[pl.ds(o_send, CHUNK), :].astype(jnp.bfloat16)
        r2 = exchange(sb2, rb2, 3, me ^ 4)
        r2.wait()
        final = keep_ref[pl.ds(o_keep, CHUNK), :] + rb2[...].astype(
            jnp.float32
        )

        own_size = CHUNK
        own_start = abs_keep + q_keep + o_keep
        out_ref[pl.ds(own_start, own_size), :] = final.astype(jnp.bfloat16)

        for s in (2, 1, 0):
            r = exchange(
                out_ref.at[pl.ds(own_start, own_size), :],
                out_ref.at[pl.ds(own_start, own_size), :],
                s, me ^ (1 << s), sems=(ag_send_sems, ag_recv_sems),
            )
            r.wait()
            bit = (me >> s) & 1
            own_start = own_start - bit * own_size
            own_size = own_size * 2

        @functools.partial(
            pl.run_scoped, second_barrier=pltpu.SemaphoreType.REGULAR
        )
        def _(second_barrier):
            for p in partners:
                pl.semaphore_signal(
                    second_barrier, inc=1,
                    device_id=(p,), device_id_type=pl.DeviceIdType.MESH,
                )
            pl.semaphore_wait(second_barrier, 3)

    out = pl.pallas_call(
        body,
        out_shape=jax.ShapeDtypeStruct((SQ, DMODEL), jnp.bfloat16),
        in_specs=[pl.BlockSpec(memory_space=pltpu.VMEM)] * 5,
        out_specs=pl.BlockSpec(memory_space=pltpu.VMEM),
        scratch_shapes=[
            pltpu.VMEM((QTR, DMODEL), jnp.float32),
            pltpu.VMEM((QTR, DMODEL), jnp.bfloat16),
            pltpu.VMEM((QTR, DMODEL), jnp.bfloat16),
            pltpu.VMEM((QTR, DMODEL), jnp.bfloat16),
            pltpu.VMEM((CHUNK, DMODEL), jnp.bfloat16),
            pltpu.VMEM((QTR, DMODEL), jnp.bfloat16),
            pltpu.VMEM((QTR, DMODEL), jnp.bfloat16),
            pltpu.VMEM((QTR, DMODEL), jnp.bfloat16),
            pltpu.VMEM((CHUNK, DMODEL), jnp.bfloat16),
            pltpu.SemaphoreType.DMA((4,)),
            pltpu.SemaphoreType.DMA((4,)),
            pltpu.SemaphoreType.DMA((3,)),
            pltpu.SemaphoreType.DMA((3,)),
        ],
        compiler_params=pltpu.CompilerParams(collective_id=0),
    )(xb, wq, k, v, wo)

    return out[None]
